# baseline (speedup 1.0000x reference)
import functools

import jax
import jax.numpy as jnp
from jax import lax
from jax.experimental import pallas as pl
from jax.experimental.pallas import tpu as pltpu
from jax.experimental.pallas import tpu_sc as plsc

N = 10000
DIM = 64
TOPK = 32
BATCH = 4
NPAD = 10240
NSLOT = 8
RA = 40
RB = 1000
RD = 1000
RN = 400
NSC = 10240
NW = 32
PER_W = NSC // NW
G = 4
NGRP = PER_W // G


def _topk_kernel(embT_ref, er_ref, idx_ref):
    R = er_ref.shape[0]
    er = er_ref[...]
    vz = jnp.full((R, 128), -3.0, jnp.float32)
    iz = jnp.zeros((R, 128), jnp.int32)

    def superchunk(c, carry):
        vs = list(carry[:NSLOT])
        js = list(carry[NSLOT:])
        cs = c * 1024
        et = embT_ref[:, pl.ds(cs, 1024)]
        s = lax.dot_general(er, et, (((1,), (0,)), ((), ())),
                            preferred_element_type=jnp.float32)
        nsq = jnp.sum(et * et, axis=0, keepdims=True)
        rn = lax.rsqrt(nsq + 1e-30)
        col = cs + lax.broadcasted_iota(jnp.int32, (R, 1024), 1)
        sv = jnp.where(col < N, s * rn, -2.0)

        for k in range(8):
            cv = sv[:, k * 128:(k + 1) * 128]
            ci = cs + k * 128 + lax.broadcasted_iota(jnp.int32, (R, 128), 1)
            for t in range(NSLOT):
                cond = cv > vs[t]
                nv = jnp.where(cond, cv, vs[t])
                ni = jnp.where(cond, ci, js[t])
                cv = jnp.where(cond, vs[t], cv)
                ci = jnp.where(cond, js[t], ci)
                vs[t] = nv
                js[t] = ni
        return tuple(vs) + tuple(js)

    carry = lax.fori_loop(0, NPAD // 1024, superchunk,
                          (vz,) * NSLOT + (iz,) * NSLOT)
    v = jnp.concatenate(carry[:NSLOT], axis=1)
    ia = jnp.concatenate(carry[NSLOT:], axis=1)

    lk = lax.broadcasted_iota(jnp.int32, (R, TOPK), 1)

    def ext(k, carry2):
        v2, res = carry2
        m = jnp.max(v2, axis=1, keepdims=True)
        eq = v2 == m
        isel = jnp.where(eq, ia, jnp.int32(2**30))
        imin = jnp.min(isel, axis=1, keepdims=True)
        v2 = jnp.where(ia == imin, -3.0, v2)
        return (v2, jnp.where(lk == k, imin, res))

    _, res = lax.fori_loop(0, TOPK, ext,
                           (v, jnp.zeros((R, TOPK), jnp.int32)))
    idx_ref[...] = res


def _topk_idx(emb):
    embT = jnp.zeros((DIM, NPAD), jnp.float32).at[:, :N].set(emb.T)
    return pl.pallas_call(
        _topk_kernel,
        grid=(N // RA,),
        in_specs=[
            pl.BlockSpec((DIM, NPAD), lambda i: (0, 0)),
            pl.BlockSpec((RA, DIM), lambda i: (i, 0)),
        ],
        out_specs=pl.BlockSpec((RA, TOPK), lambda i: (i, 0)),
        out_shape=jax.ShapeDtypeStruct((N, TOPK), jnp.int32),
    )(embT, emb)


def _tables_kernel(data_ref, ef_ref, eb_ref, wu_ref, evf_ref, evb_ref,
                   xtf_ref, xtb_ref, aif_ref, aib_ref):
    R = ef_ref.shape[0]
    ef2 = jnp.dot(ef_ref[...], evf_ref[...],
                  preferred_element_type=jnp.float32)
    eb2 = jnp.dot(eb_ref[...], evb_ref[...],
                  preferred_element_type=jnp.float32)
    jf, if_, jb, ib = [], [], [], []
    for b in range(BATCH):
        t = jnp.dot(data_ref[b], wu_ref[...],
                    preferred_element_type=jnp.float32)
        xtf_ref[:, b * 64:(b + 1) * 64] = t[:, :64]
        xtb_ref[:, b * 64:(b + 1) * 64] = t[:, 64:128]
        jf.append(t[:, 128:129] + ef2[:, 0:1])
        if_.append(t[:, 129:130] + ef2[:, 1:2])
        jb.append(t[:, 130:131] + eb2[:, 0:1])
        ib.append(t[:, 131:132] + eb2[:, 1:2])
    z12 = jnp.zeros((R, 12), jnp.float32)
    xtf_ref[:, 256:272] = jnp.concatenate(jf + [z12], axis=1)
    xtb_ref[:, 256:272] = jnp.concatenate(jb + [z12], axis=1)
    aif_ref[...] = jnp.concatenate(if_ + [z12], axis=1)
    aib_ref[...] = jnp.concatenate(ib + [z12], axis=1)


def _build_tables(data, emb_f, emb_b, W_f, att_i_f, att_j_f,
                  W_b, att_i_b, att_j_b):
    Wbf = jnp.flip(W_b, axis=0)
    wu = jnp.zeros((10, 256), jnp.float32)
    wu = wu.at[:, :64].set(W_f).at[:, 64:128].set(Wbf)
    wu = wu.at[:, 128].set(W_f @ att_j_f[:DIM])
    wu = wu.at[:, 129].set(W_f @ att_i_f[:DIM])
    wu = wu.at[:, 130].set(Wbf @ att_j_b[:DIM])
    wu = wu.at[:, 131].set(Wbf @ att_i_b[:DIM])
    evf = jnp.stack([att_j_f[DIM:], att_i_f[DIM:]], axis=1)
    evb = jnp.stack([att_j_b[DIM:], att_i_b[DIM:]], axis=1)
    return pl.pallas_call(
        _tables_kernel,
        grid=(N // RB,),
        in_specs=[
            pl.BlockSpec((BATCH, RB, 10), lambda i: (0, i, 0)),
            pl.BlockSpec((RB, DIM), lambda i: (i, 0)),
            pl.BlockSpec((RB, DIM), lambda i: (i, 0)),
            pl.BlockSpec((10, 256), lambda i: (0, 0)),
            pl.BlockSpec((DIM, 2), lambda i: (0, 0)),
            pl.BlockSpec((DIM, 2), lambda i: (0, 0)),
        ],
        out_specs=[
            pl.BlockSpec((RB, 384), lambda i: (i, 0)),
            pl.BlockSpec((RB, 384), lambda i: (i, 0)),
            pl.BlockSpec((RB, 16), lambda i: (i, 0)),
            pl.BlockSpec((RB, 16), lambda i: (i, 0)),
        ],
        out_shape=[
            jax.ShapeDtypeStruct((NSC, 384), jnp.float32),
            jax.ShapeDtypeStruct((NSC, 384), jnp.float32),
            jax.ShapeDtypeStruct((NSC, 16), jnp.float32),
            jax.ShapeDtypeStruct((NSC, 16), jnp.float32),
        ],
    )(data, emb_f, emb_b, wu, evf, evb)


def _lane_bcast(v, lane):
    dn = lax.GatherDimensionNumbers(offset_dims=(), collapsed_slice_dims=(0,),
                                    start_index_map=(0,))
    idx = jnp.full((16, 1), lane, jnp.int32)
    return lax.gather(v, idx, dn, (1,),
                      mode=lax.GatherScatterMode.PROMISE_IN_BOUNDS)


def _sc_agg_body(xt, ai, idx, out_hbm,
                 idx_v, ai_v, rows_v, al_v, ostg_v, sem1):
    w = lax.axis_index("c") * 16 + lax.axis_index("s")
    pltpu.sync_copy(idx.at[pl.ds(w * PER_W * TOPK, PER_W * TOPK)], idx_v)
    pltpu.sync_copy(ai.at[pl.ds(w * PER_W * 16, PER_W * 16)], ai_v)

    def group(g, _):
        ixs = idx_v.at[pl.ds(g * (G * TOPK), G * TOPK)]
        pltpu.async_copy(xt.at[ixs], rows_v, sem1).wait()

        for ii in range(G):
            r0 = ii * TOPK
            nl = g * G + ii
            ai_vec = ai_v[pl.ds(nl * 16, 16)]

            def p1(k, m):
                alk = ai_vec + rows_v[r0 + k, pl.ds(256, 16)]
                alk = jnp.where(alk > 0, alk, 0.2 * alk)
                al_v[pl.ds(k * 16, 16)] = alk
                return jnp.maximum(m, alk)

            m = lax.fori_loop(0, TOPK, p1,
                              jnp.full((16,), -3.0e38, jnp.float32))

            def p2(k, den):
                e = jnp.exp(al_v[pl.ds(k * 16, 16)] - m)
                al_v[pl.ds(k * 16, 16)] = e
                return den + e

            den = lax.fori_loop(0, TOPK, p2, jnp.zeros((16,), jnp.float32))
            rden = 1.0 / (den + 1e-16)

            def p3(k, accs):
                wt = al_v[pl.ds(k * 16, 16)] * rden
                accs = list(accs)
                for b in range(BATCH):
                    wb = _lane_bcast(wt, b)
                    boff = b * 64
                    for cc in range(4):
                        accs[b * 4 + cc] = accs[b * 4 + cc] + wb * rows_v[
                            r0 + k, pl.ds(boff + cc * 16, 16)]
                return tuple(accs)

            z = jnp.zeros((16,), jnp.float32)
            accs = lax.fori_loop(0, TOPK, p3, (z,) * 16)
            for b in range(BATCH):
                for cc in range(4):
                    ostg_v[pl.ds(ii * 256 + b * 64 + cc * 16, 16)] = (
                        accs[b * 4 + cc])

        base = (w * PER_W + g * G) * 256
        pltpu.sync_copy(ostg_v, out_hbm.at[pl.ds(base, G * 256)])
        return 0
    lax.fori_loop(0, NGRP, group, 0)


def _sc_aggregate_dir(xtab, ai, idx):
    mesh = plsc.VectorSubcoreMesh(core_axis_name="c", subcore_axis_name="s")
    kern = functools.partial(
        pl.kernel,
        out_type=jax.ShapeDtypeStruct((NSC * 256,), jnp.float32),
        mesh=mesh,
        scratch_types=[
            pltpu.VMEM((PER_W * TOPK,), jnp.int32),
            pltpu.VMEM((PER_W * 16,), jnp.float32),
            pltpu.VMEM((G * TOPK, 384), jnp.float32),
            pltpu.VMEM((TOPK * 16,), jnp.float32),
            pltpu.VMEM((G * 256,), jnp.float32),
            pltpu.SemaphoreType.DMA,
        ],
    )(_sc_agg_body)
    idx_p = jnp.zeros((NSC, TOPK), jnp.int32).at[:N].set(idx)
    return kern(xtab, ai.reshape(NSC * 16), idx_p.reshape(NSC * TOPK))


def _stats_kernel(x_ref, sum_ref, sq_ref):
    i = pl.program_id(0)
    x = x_ref[...]
    pad = jnp.zeros((7, 256), jnp.float32)
    cs = jnp.concatenate([jnp.sum(x, axis=0, keepdims=True), pad], axis=0)
    cq = jnp.concatenate([jnp.sum(x * x, axis=0, keepdims=True), pad], axis=0)
    first = (i % (N // RD)) == 0

    @pl.when(first)
    def _():
        sum_ref[...] = cs
        sq_ref[...] = cq

    @pl.when(jnp.logical_not(first))
    def _():
        sum_ref[...] = sum_ref[...] + cs
        sq_ref[...] = sq_ref[...] + cq


def _finish_kernel(xf_ref, xb_ref, sc_ref, sh_ref, o_ref):
    scale_f = sc_ref[0:1, :]
    scale_b = sc_ref[1:2, :]
    shift_f = sh_ref[0:1, :]
    shift_b = sh_ref[1:2, :]
    for b in range(BATCH):
        f = xf_ref[:, b * 64:(b + 1) * 64] * scale_f + shift_f
        bb = xb_ref[:, b * 64:(b + 1) * 64] * scale_b + shift_b
        f = jnp.maximum(f, 0.0)
        bb = jnp.maximum(bb, 0.0)
        y = jnp.concatenate([f, bb], axis=-1)
        o_ref[b] = jnp.where(y > 0, y, 0.01 * y)


def _finish(outagg, gamma_f, beta_f, gamma_b, beta_b, b_f, b_b):
    sums, sqs = pl.pallas_call(
        _stats_kernel,
        grid=(2 * N // RD,),
        in_specs=[pl.BlockSpec((RD, 256), lambda i: (i, 0))],
        out_specs=[
            pl.BlockSpec((8, 256), lambda i: ((i * RD) // N, 0)),
            pl.BlockSpec((8, 256), lambda i: ((i * RD) // N, 0)),
        ],
        out_shape=[
            jax.ShapeDtypeStruct((16, 256), jnp.float32),
            jax.ShapeDtypeStruct((16, 256), jnp.float32),
        ],
    )(outagg)
    sums = sums.reshape(2, 8, 256)[:, 0, :]
    sqs = sqs.reshape(2, 8, 256)[:, 0, :]
    cnt = BATCH * N
    del b_f, b_b
    mean = sums.reshape(2, 4, 64).sum(axis=1) / cnt
    ex2 = sqs.reshape(2, 4, 64).sum(axis=1) / cnt
    var = ex2 - mean * mean
    rstd = lax.rsqrt(var + 1e-5)
    gam = jnp.stack([gamma_f, gamma_b], axis=0)
    bet = jnp.stack([beta_f, beta_b], axis=0)
    scale = rstd * gam
    shift2 = bet - mean * scale
    return pl.pallas_call(
        _finish_kernel,
        grid=(N // RN,),
        in_specs=[
            pl.BlockSpec((RN, 256), lambda i: (i, 0)),
            pl.BlockSpec((RN, 256), lambda i: (i + N // RN, 0)),
            pl.BlockSpec((2, 64), lambda i: (0, 0)),
            pl.BlockSpec((2, 64), lambda i: (0, 0)),
        ],
        out_specs=pl.BlockSpec((BATCH, RN, 128), lambda i: (0, i, 0)),
        out_shape=jax.ShapeDtypeStruct((BATCH, N, 128), jnp.float32),
    )(outagg, outagg, scale, shift2)


def kernel(data, org_edge_index, emb_f, emb_b, W_f, b_f, att_i_f, att_j_f,
           gamma_f, beta_f, W_b, b_b, att_i_b, att_j_b, gamma_b, beta_b):
    xtabf, xtabb, aif, aib = _build_tables(
        data, emb_f, emb_b, W_f, att_i_f, att_j_f, W_b, att_i_b, att_j_b)
    idx_f = _topk_idx(emb_f)
    of = _sc_aggregate_dir(xtabf, aif, idx_f)
    idx_b = _topk_idx(emb_b)
    ob = _sc_aggregate_dir(xtabb, aib, idx_b)
    outagg = jnp.concatenate(
        [of.reshape(NSC, 256)[:N], ob.reshape(NSC, 256)[:N]], axis=0)
    return _finish(outagg, gamma_f, beta_f, gamma_b, beta_b, b_f, b_b)


# speedup vs baseline: 1.7559x; 1.7559x over previous
import functools

import jax
import jax.numpy as jnp
from jax import lax
from jax.experimental import pallas as pl
from jax.experimental.pallas import tpu as pltpu
from jax.experimental.pallas import tpu_sc as plsc

N = 10000
DIM = 64
TOPK = 32
BATCH = 4
NPAD = 10240
NSLOT = 8
RA = 200
RB = 1000
RD = 1000
RN = 400
NSC = 10240
NW = 32
PER_W = NSC // NW
G = 4
NGRP = PER_W // G


def _topk_kernel(embT_ref, er_ref, idx_ref):
    R = er_ref.shape[0]
    er = er_ref[...]
    vz = jnp.full((R, 128), -3.0, jnp.float32)
    iz = jnp.zeros((R, 128), jnp.int32)

    def superchunk(c, carry):
        vs = list(carry[:NSLOT])
        js = list(carry[NSLOT:])
        cs = c * 1024
        et = embT_ref[:, pl.ds(cs, 1024)]
        s = lax.dot_general(er, et, (((1,), (0,)), ((), ())),
                            preferred_element_type=jnp.float32)
        nsq = jnp.sum(et * et, axis=0, keepdims=True)
        rn = lax.rsqrt(nsq + 1e-30)
        col = cs + lax.broadcasted_iota(jnp.int32, (R, 1024), 1)
        sv = jnp.where(col < N, s * rn, -2.0)

        for k in range(8):
            cv = sv[:, k * 128:(k + 1) * 128]
            ci = cs + k * 128 + lax.broadcasted_iota(jnp.int32, (R, 128), 1)
            for t in range(NSLOT):
                cond = cv > vs[t]
                nv = jnp.where(cond, cv, vs[t])
                ni = jnp.where(cond, ci, js[t])
                cv = jnp.where(cond, vs[t], cv)
                ci = jnp.where(cond, js[t], ci)
                vs[t] = nv
                js[t] = ni
        return tuple(vs) + tuple(js)

    carry = lax.fori_loop(0, NPAD // 1024, superchunk,
                          (vz,) * NSLOT + (iz,) * NSLOT)
    v = jnp.concatenate(carry[:NSLOT], axis=1)
    ia = jnp.concatenate(carry[NSLOT:], axis=1)

    lk = lax.broadcasted_iota(jnp.int32, (R, TOPK), 1)

    def ext(k, carry2):
        v2, res = carry2
        m = jnp.max(v2, axis=1, keepdims=True)
        eq = v2 == m
        isel = jnp.where(eq, ia, jnp.int32(2**30))
        imin = jnp.min(isel, axis=1, keepdims=True)
        v2 = jnp.where(ia == imin, -3.0, v2)
        return (v2, jnp.where(lk == k, imin, res))

    _, res = lax.fori_loop(0, TOPK, ext,
                           (v, jnp.zeros((R, TOPK), jnp.int32)))
    idx_ref[...] = res


def _topk_idx(emb):
    embT = jnp.zeros((DIM, NPAD), jnp.float32).at[:, :N].set(emb.T)
    return pl.pallas_call(
        _topk_kernel,
        grid=(N // RA,),
        in_specs=[
            pl.BlockSpec((DIM, NPAD), lambda i: (0, 0)),
            pl.BlockSpec((RA, DIM), lambda i: (i, 0)),
        ],
        out_specs=pl.BlockSpec((RA, TOPK), lambda i: (i, 0)),
        out_shape=jax.ShapeDtypeStruct((N, TOPK), jnp.int32),
    )(embT, emb)


def _tables_kernel(data_ref, ef_ref, eb_ref, wu_ref, evf_ref, evb_ref,
                   xtf_ref, xtb_ref, aif_ref, aib_ref):
    R = ef_ref.shape[0]
    ef2 = jnp.dot(ef_ref[...], evf_ref[...],
                  preferred_element_type=jnp.float32)
    eb2 = jnp.dot(eb_ref[...], evb_ref[...],
                  preferred_element_type=jnp.float32)
    jf, if_, jb, ib = [], [], [], []
    for b in range(BATCH):
        t = jnp.dot(data_ref[b], wu_ref[...],
                    preferred_element_type=jnp.float32)
        xtf_ref[:, b * 64:(b + 1) * 64] = t[:, :64]
        xtb_ref[:, b * 64:(b + 1) * 64] = t[:, 64:128]
        jf.append(t[:, 128:129] + ef2[:, 0:1])
        if_.append(t[:, 129:130] + ef2[:, 1:2])
        jb.append(t[:, 130:131] + eb2[:, 0:1])
        ib.append(t[:, 131:132] + eb2[:, 1:2])
    z12 = jnp.zeros((R, 12), jnp.float32)
    xtf_ref[:, 256:272] = jnp.concatenate(jf + [z12], axis=1)
    xtb_ref[:, 256:272] = jnp.concatenate(jb + [z12], axis=1)
    aif_ref[...] = jnp.concatenate(if_ + [z12], axis=1)
    aib_ref[...] = jnp.concatenate(ib + [z12], axis=1)


def _build_tables(data, emb_f, emb_b, W_f, att_i_f, att_j_f,
                  W_b, att_i_b, att_j_b):
    Wbf = jnp.flip(W_b, axis=0)
    wu = jnp.zeros((10, 256), jnp.float32)
    wu = wu.at[:, :64].set(W_f).at[:, 64:128].set(Wbf)
    wu = wu.at[:, 128].set(W_f @ att_j_f[:DIM])
    wu = wu.at[:, 129].set(W_f @ att_i_f[:DIM])
    wu = wu.at[:, 130].set(Wbf @ att_j_b[:DIM])
    wu = wu.at[:, 131].set(Wbf @ att_i_b[:DIM])
    evf = jnp.stack([att_j_f[DIM:], att_i_f[DIM:]], axis=1)
    evb = jnp.stack([att_j_b[DIM:], att_i_b[DIM:]], axis=1)
    return pl.pallas_call(
        _tables_kernel,
        grid=(N // RB,),
        in_specs=[
            pl.BlockSpec((BATCH, RB, 10), lambda i: (0, i, 0)),
            pl.BlockSpec((RB, DIM), lambda i: (i, 0)),
            pl.BlockSpec((RB, DIM), lambda i: (i, 0)),
            pl.BlockSpec((10, 256), lambda i: (0, 0)),
            pl.BlockSpec((DIM, 2), lambda i: (0, 0)),
            pl.BlockSpec((DIM, 2), lambda i: (0, 0)),
        ],
        out_specs=[
            pl.BlockSpec((RB, 384), lambda i: (i, 0)),
            pl.BlockSpec((RB, 384), lambda i: (i, 0)),
            pl.BlockSpec((RB, 16), lambda i: (i, 0)),
            pl.BlockSpec((RB, 16), lambda i: (i, 0)),
        ],
        out_shape=[
            jax.ShapeDtypeStruct((NSC, 384), jnp.float32),
            jax.ShapeDtypeStruct((NSC, 384), jnp.float32),
            jax.ShapeDtypeStruct((NSC, 16), jnp.float32),
            jax.ShapeDtypeStruct((NSC, 16), jnp.float32),
        ],
    )(data, emb_f, emb_b, wu, evf, evb)


def _lane_bcast(v, lane):
    dn = lax.GatherDimensionNumbers(offset_dims=(), collapsed_slice_dims=(0,),
                                    start_index_map=(0,))
    idx = jnp.full((16, 1), lane, jnp.int32)
    return lax.gather(v, idx, dn, (1,),
                      mode=lax.GatherScatterMode.PROMISE_IN_BOUNDS)


def _sc_agg_body(xt, ai, idx, out_hbm,
                 idx_v, ai_v, rows_v, al_v, ostg_v, sem1):
    w = lax.axis_index("c") * 16 + lax.axis_index("s")
    pltpu.sync_copy(idx.at[pl.ds(w * PER_W * TOPK, PER_W * TOPK)], idx_v)
    pltpu.sync_copy(ai.at[pl.ds(w * PER_W * 16, PER_W * 16)], ai_v)

    def group(g, _):
        ixs = idx_v.at[pl.ds(g * (G * TOPK), G * TOPK)]
        pltpu.async_copy(xt.at[ixs], rows_v, sem1).wait()

        for ii in range(G):
            r0 = ii * TOPK
            nl = g * G + ii
            ai_vec = ai_v[pl.ds(nl * 16, 16)]

            def p1(k, m):
                alk = ai_vec + rows_v[r0 + k, pl.ds(256, 16)]
                alk = jnp.where(alk > 0, alk, 0.2 * alk)
                al_v[pl.ds(k * 16, 16)] = alk
                return jnp.maximum(m, alk)

            m = lax.fori_loop(0, TOPK, p1,
                              jnp.full((16,), -3.0e38, jnp.float32))

            def p2(k, den):
                e = jnp.exp(al_v[pl.ds(k * 16, 16)] - m)
                al_v[pl.ds(k * 16, 16)] = e
                return den + e

            den = lax.fori_loop(0, TOPK, p2, jnp.zeros((16,), jnp.float32))
            rden = 1.0 / (den + 1e-16)

            def p3(k, accs):
                wt = al_v[pl.ds(k * 16, 16)] * rden
                accs = list(accs)
                for b in range(BATCH):
                    wb = _lane_bcast(wt, b)
                    boff = b * 64
                    for cc in range(4):
                        accs[b * 4 + cc] = accs[b * 4 + cc] + wb * rows_v[
                            r0 + k, pl.ds(boff + cc * 16, 16)]
                return tuple(accs)

            z = jnp.zeros((16,), jnp.float32)
            accs = lax.fori_loop(0, TOPK, p3, (z,) * 16)
            for b in range(BATCH):
                for cc in range(4):
                    ostg_v[pl.ds(ii * 256 + b * 64 + cc * 16, 16)] = (
                        accs[b * 4 + cc])

        base = (w * PER_W + g * G) * 256
        pltpu.sync_copy(ostg_v, out_hbm.at[pl.ds(base, G * 256)])
        return 0
    lax.fori_loop(0, NGRP, group, 0)


def _sc_aggregate_dir(xtab, ai, idx):
    mesh = plsc.VectorSubcoreMesh(core_axis_name="c", subcore_axis_name="s")
    kern = functools.partial(
        pl.kernel,
        out_type=jax.ShapeDtypeStruct((NSC * 256,), jnp.float32),
        mesh=mesh,
        scratch_types=[
            pltpu.VMEM((PER_W * TOPK,), jnp.int32),
            pltpu.VMEM((PER_W * 16,), jnp.float32),
            pltpu.VMEM((G * TOPK, 384), jnp.float32),
            pltpu.VMEM((TOPK * 16,), jnp.float32),
            pltpu.VMEM((G * 256,), jnp.float32),
            pltpu.SemaphoreType.DMA,
        ],
    )(_sc_agg_body)
    idx_p = jnp.zeros((NSC, TOPK), jnp.int32).at[:N].set(idx)
    return kern(xtab, ai.reshape(NSC * 16), idx_p.reshape(NSC * TOPK))


def _stats_kernel(x_ref, sum_ref, sq_ref):
    i = pl.program_id(0)
    x = x_ref[...]
    pad = jnp.zeros((7, 256), jnp.float32)
    cs = jnp.concatenate([jnp.sum(x, axis=0, keepdims=True), pad], axis=0)
    cq = jnp.concatenate([jnp.sum(x * x, axis=0, keepdims=True), pad], axis=0)
    first = (i % (N // RD)) == 0

    @pl.when(first)
    def _():
        sum_ref[...] = cs
        sq_ref[...] = cq

    @pl.when(jnp.logical_not(first))
    def _():
        sum_ref[...] = sum_ref[...] + cs
        sq_ref[...] = sq_ref[...] + cq


def _finish_kernel(xf_ref, xb_ref, sc_ref, sh_ref, o_ref):
    scale_f = sc_ref[0:1, :]
    scale_b = sc_ref[1:2, :]
    shift_f = sh_ref[0:1, :]
    shift_b = sh_ref[1:2, :]
    for b in range(BATCH):
        f = xf_ref[:, b * 64:(b + 1) * 64] * scale_f + shift_f
        bb = xb_ref[:, b * 64:(b + 1) * 64] * scale_b + shift_b
        f = jnp.maximum(f, 0.0)
        bb = jnp.maximum(bb, 0.0)
        y = jnp.concatenate([f, bb], axis=-1)
        o_ref[b] = jnp.where(y > 0, y, 0.01 * y)


def _finish(outagg, gamma_f, beta_f, gamma_b, beta_b, b_f, b_b):
    sums, sqs = pl.pallas_call(
        _stats_kernel,
        grid=(2 * N // RD,),
        in_specs=[pl.BlockSpec((RD, 256), lambda i: (i, 0))],
        out_specs=[
            pl.BlockSpec((8, 256), lambda i: ((i * RD) // N, 0)),
            pl.BlockSpec((8, 256), lambda i: ((i * RD) // N, 0)),
        ],
        out_shape=[
            jax.ShapeDtypeStruct((16, 256), jnp.float32),
            jax.ShapeDtypeStruct((16, 256), jnp.float32),
        ],
    )(outagg)
    sums = sums.reshape(2, 8, 256)[:, 0, :]
    sqs = sqs.reshape(2, 8, 256)[:, 0, :]
    cnt = BATCH * N
    del b_f, b_b
    mean = sums.reshape(2, 4, 64).sum(axis=1) / cnt
    ex2 = sqs.reshape(2, 4, 64).sum(axis=1) / cnt
    var = ex2 - mean * mean
    rstd = lax.rsqrt(var + 1e-5)
    gam = jnp.stack([gamma_f, gamma_b], axis=0)
    bet = jnp.stack([beta_f, beta_b], axis=0)
    scale = rstd * gam
    shift2 = bet - mean * scale
    return pl.pallas_call(
        _finish_kernel,
        grid=(N // RN,),
        in_specs=[
            pl.BlockSpec((RN, 256), lambda i: (i, 0)),
            pl.BlockSpec((RN, 256), lambda i: (i + N // RN, 0)),
            pl.BlockSpec((2, 64), lambda i: (0, 0)),
            pl.BlockSpec((2, 64), lambda i: (0, 0)),
        ],
        out_specs=pl.BlockSpec((BATCH, RN, 128), lambda i: (0, i, 0)),
        out_shape=jax.ShapeDtypeStruct((BATCH, N, 128), jnp.float32),
    )(outagg, outagg, scale, shift2)


def kernel(data, org_edge_index, emb_f, emb_b, W_f, b_f, att_i_f, att_j_f,
           gamma_f, beta_f, W_b, b_b, att_i_b, att_j_b, gamma_b, beta_b):
    xtabf, xtabb, aif, aib = _build_tables(
        data, emb_f, emb_b, W_f, att_i_f, att_j_f, W_b, att_i_b, att_j_b)
    idx_f = _topk_idx(emb_f)
    of = _sc_aggregate_dir(xtabf, aif, idx_f)
    idx_b = _topk_idx(emb_b)
    ob = _sc_aggregate_dir(xtabb, aib, idx_b)
    outagg = jnp.concatenate(
        [of.reshape(NSC, 256)[:N], ob.reshape(NSC, 256)[:N]], axis=0)
    return _finish(outagg, gamma_f, beta_f, gamma_b, beta_b, b_f, b_b)


# speedup vs baseline: 1.8644x; 1.0618x over previous
import functools

import jax
import jax.numpy as jnp
from jax import lax
from jax.experimental import pallas as pl
from jax.experimental.pallas import tpu as pltpu
from jax.experimental.pallas import tpu_sc as plsc

N = 10000
DIM = 64
TOPK = 32
BATCH = 4
NPAD = 10240
NSLOT = 8
RA = 400
RB = 1000
RD = 1000
RN = 400
NSC = 10240
NW = 32
PER_W = NSC // NW
G = 4
NGRP = PER_W // G


def _topk_kernel(embT_ref, er_ref, idx_ref):
    R = er_ref.shape[0]
    er = er_ref[...]
    vz = jnp.full((R, 128), -3.0, jnp.float32)
    iz = jnp.zeros((R, 128), jnp.int32)

    def superchunk(c, carry):
        vs = list(carry[:NSLOT])
        js = list(carry[NSLOT:])
        cs = c * 1024
        et = embT_ref[:, pl.ds(cs, 1024)]
        s = lax.dot_general(er, et, (((1,), (0,)), ((), ())),
                            preferred_element_type=jnp.float32)
        nsq = jnp.sum(et * et, axis=0, keepdims=True)
        rn = lax.rsqrt(nsq + 1e-30)
        col = cs + lax.broadcasted_iota(jnp.int32, (R, 1024), 1)
        sv = jnp.where(col < N, s * rn, -2.0)

        for k in range(8):
            cv = sv[:, k * 128:(k + 1) * 128]
            ci = cs + k * 128 + lax.broadcasted_iota(jnp.int32, (R, 128), 1)
            for t in range(NSLOT):
                cond = cv > vs[t]
                nv = jnp.where(cond, cv, vs[t])
                ni = jnp.where(cond, ci, js[t])
                cv = jnp.where(cond, vs[t], cv)
                ci = jnp.where(cond, js[t], ci)
                vs[t] = nv
                js[t] = ni
        return tuple(vs) + tuple(js)

    carry = lax.fori_loop(0, NPAD // 1024, superchunk,
                          (vz,) * NSLOT + (iz,) * NSLOT)
    v = jnp.concatenate(carry[:NSLOT], axis=1)
    ia = jnp.concatenate(carry[NSLOT:], axis=1)

    lk = lax.broadcasted_iota(jnp.int32, (R, TOPK), 1)

    def ext(k, carry2):
        v2, res = carry2
        m = jnp.max(v2, axis=1, keepdims=True)
        eq = v2 == m
        isel = jnp.where(eq, ia, jnp.int32(2**30))
        imin = jnp.min(isel, axis=1, keepdims=True)
        v2 = jnp.where(ia == imin, -3.0, v2)
        return (v2, jnp.where(lk == k, imin, res))

    _, res = lax.fori_loop(0, TOPK, ext,
                           (v, jnp.zeros((R, TOPK), jnp.int32)))
    idx_ref[...] = res


def _topk_idx(emb):
    embT = jnp.zeros((DIM, NPAD), jnp.float32).at[:, :N].set(emb.T)
    return pl.pallas_call(
        _topk_kernel,
        grid=(N // RA,),
        in_specs=[
            pl.BlockSpec((DIM, NPAD), lambda i: (0, 0)),
            pl.BlockSpec((RA, DIM), lambda i: (i, 0)),
        ],
        out_specs=pl.BlockSpec((RA, TOPK), lambda i: (i, 0)),
        out_shape=jax.ShapeDtypeStruct((N, TOPK), jnp.int32),
    )(embT, emb)


def _tables_kernel(data_ref, ef_ref, eb_ref, wu_ref, evf_ref, evb_ref,
                   xtf_ref, xtb_ref, aif_ref, aib_ref):
    R = ef_ref.shape[0]
    ef2 = jnp.dot(ef_ref[...], evf_ref[...],
                  preferred_element_type=jnp.float32)
    eb2 = jnp.dot(eb_ref[...], evb_ref[...],
                  preferred_element_type=jnp.float32)
    jf, if_, jb, ib = [], [], [], []
    for b in range(BATCH):
        t = jnp.dot(data_ref[b], wu_ref[...],
                    preferred_element_type=jnp.float32)
        xtf_ref[:, b * 64:(b + 1) * 64] = t[:, :64]
        xtb_ref[:, b * 64:(b + 1) * 64] = t[:, 64:128]
        jf.append(t[:, 128:129] + ef2[:, 0:1])
        if_.append(t[:, 129:130] + ef2[:, 1:2])
        jb.append(t[:, 130:131] + eb2[:, 0:1])
        ib.append(t[:, 131:132] + eb2[:, 1:2])
    z12 = jnp.zeros((R, 12), jnp.float32)
    xtf_ref[:, 256:272] = jnp.concatenate(jf + [z12], axis=1)
    xtb_ref[:, 256:272] = jnp.concatenate(jb + [z12], axis=1)
    aif_ref[...] = jnp.concatenate(if_ + [z12], axis=1)
    aib_ref[...] = jnp.concatenate(ib + [z12], axis=1)


def _build_tables(data, emb_f, emb_b, W_f, att_i_f, att_j_f,
                  W_b, att_i_b, att_j_b):
    Wbf = jnp.flip(W_b, axis=0)
    wu = jnp.zeros((10, 256), jnp.float32)
    wu = wu.at[:, :64].set(W_f).at[:, 64:128].set(Wbf)
    wu = wu.at[:, 128].set(W_f @ att_j_f[:DIM])
    wu = wu.at[:, 129].set(W_f @ att_i_f[:DIM])
    wu = wu.at[:, 130].set(Wbf @ att_j_b[:DIM])
    wu = wu.at[:, 131].set(Wbf @ att_i_b[:DIM])
    evf = jnp.stack([att_j_f[DIM:], att_i_f[DIM:]], axis=1)
    evb = jnp.stack([att_j_b[DIM:], att_i_b[DIM:]], axis=1)
    return pl.pallas_call(
        _tables_kernel,
        grid=(N // RB,),
        in_specs=[
            pl.BlockSpec((BATCH, RB, 10), lambda i: (0, i, 0)),
            pl.BlockSpec((RB, DIM), lambda i: (i, 0)),
            pl.BlockSpec((RB, DIM), lambda i: (i, 0)),
            pl.BlockSpec((10, 256), lambda i: (0, 0)),
            pl.BlockSpec((DIM, 2), lambda i: (0, 0)),
            pl.BlockSpec((DIM, 2), lambda i: (0, 0)),
        ],
        out_specs=[
            pl.BlockSpec((RB, 384), lambda i: (i, 0)),
            pl.BlockSpec((RB, 384), lambda i: (i, 0)),
            pl.BlockSpec((RB, 16), lambda i: (i, 0)),
            pl.BlockSpec((RB, 16), lambda i: (i, 0)),
        ],
        out_shape=[
            jax.ShapeDtypeStruct((NSC, 384), jnp.float32),
            jax.ShapeDtypeStruct((NSC, 384), jnp.float32),
            jax.ShapeDtypeStruct((NSC, 16), jnp.float32),
            jax.ShapeDtypeStruct((NSC, 16), jnp.float32),
        ],
    )(data, emb_f, emb_b, wu, evf, evb)


def _lane_bcast(v, lane):
    dn = lax.GatherDimensionNumbers(offset_dims=(), collapsed_slice_dims=(0,),
                                    start_index_map=(0,))
    idx = jnp.full((16, 1), lane, jnp.int32)
    return lax.gather(v, idx, dn, (1,),
                      mode=lax.GatherScatterMode.PROMISE_IN_BOUNDS)


def _sc_agg_body(xt, ai, idx, out_hbm,
                 idx_v, ai_v, rows_v, al_v, ostg_v, sem1):
    w = lax.axis_index("c") * 16 + lax.axis_index("s")
    pltpu.sync_copy(idx.at[pl.ds(w * PER_W * TOPK, PER_W * TOPK)], idx_v)
    pltpu.sync_copy(ai.at[pl.ds(w * PER_W * 16, PER_W * 16)], ai_v)

    def group(g, _):
        ixs = idx_v.at[pl.ds(g * (G * TOPK), G * TOPK)]
        pltpu.async_copy(xt.at[ixs], rows_v, sem1).wait()

        for ii in range(G):
            r0 = ii * TOPK
            nl = g * G + ii
            ai_vec = ai_v[pl.ds(nl * 16, 16)]

            def p1(k, m):
                alk = ai_vec + rows_v[r0 + k, pl.ds(256, 16)]
                alk = jnp.where(alk > 0, alk, 0.2 * alk)
                al_v[pl.ds(k * 16, 16)] = alk
                return jnp.maximum(m, alk)

            m = lax.fori_loop(0, TOPK, p1,
                              jnp.full((16,), -3.0e38, jnp.float32))

            def p2(k, den):
                e = jnp.exp(al_v[pl.ds(k * 16, 16)] - m)
                al_v[pl.ds(k * 16, 16)] = e
                return den + e

            den = lax.fori_loop(0, TOPK, p2, jnp.zeros((16,), jnp.float32))
            rden = 1.0 / (den + 1e-16)

            def p3(k, accs):
                wt = al_v[pl.ds(k * 16, 16)] * rden
                accs = list(accs)
                for b in range(BATCH):
                    wb = _lane_bcast(wt, b)
                    boff = b * 64
                    for cc in range(4):
                        accs[b * 4 + cc] = accs[b * 4 + cc] + wb * rows_v[
                            r0 + k, pl.ds(boff + cc * 16, 16)]
                return tuple(accs)

            z = jnp.zeros((16,), jnp.float32)
            accs = lax.fori_loop(0, TOPK, p3, (z,) * 16)
            for b in range(BATCH):
                for cc in range(4):
                    ostg_v[pl.ds(ii * 256 + b * 64 + cc * 16, 16)] = (
                        accs[b * 4 + cc])

        base = (w * PER_W + g * G) * 256
        pltpu.sync_copy(ostg_v, out_hbm.at[pl.ds(base, G * 256)])
        return 0
    lax.fori_loop(0, NGRP, group, 0)


def _sc_aggregate_dir(xtab, ai, idx):
    mesh = plsc.VectorSubcoreMesh(core_axis_name="c", subcore_axis_name="s")
    kern = functools.partial(
        pl.kernel,
        out_type=jax.ShapeDtypeStruct((NSC * 256,), jnp.float32),
        mesh=mesh,
        scratch_types=[
            pltpu.VMEM((PER_W * TOPK,), jnp.int32),
            pltpu.VMEM((PER_W * 16,), jnp.float32),
            pltpu.VMEM((G * TOPK, 384), jnp.float32),
            pltpu.VMEM((TOPK * 16,), jnp.float32),
            pltpu.VMEM((G * 256,), jnp.float32),
            pltpu.SemaphoreType.DMA,
        ],
    )(_sc_agg_body)
    idx_p = jnp.zeros((NSC, TOPK), jnp.int32).at[:N].set(idx)
    return kern(xtab, ai.reshape(NSC * 16), idx_p.reshape(NSC * TOPK))


def _stats_kernel(x_ref, sum_ref, sq_ref):
    i = pl.program_id(0)
    x = x_ref[...]
    pad = jnp.zeros((7, 256), jnp.float32)
    cs = jnp.concatenate([jnp.sum(x, axis=0, keepdims=True), pad], axis=0)
    cq = jnp.concatenate([jnp.sum(x * x, axis=0, keepdims=True), pad], axis=0)
    first = (i % (N // RD)) == 0

    @pl.when(first)
    def _():
        sum_ref[...] = cs
        sq_ref[...] = cq

    @pl.when(jnp.logical_not(first))
    def _():
        sum_ref[...] = sum_ref[...] + cs
        sq_ref[...] = sq_ref[...] + cq


def _finish_kernel(xf_ref, xb_ref, sc_ref, sh_ref, o_ref):
    scale_f = sc_ref[0:1, :]
    scale_b = sc_ref[1:2, :]
    shift_f = sh_ref[0:1, :]
    shift_b = sh_ref[1:2, :]
    for b in range(BATCH):
        f = xf_ref[:, b * 64:(b + 1) * 64] * scale_f + shift_f
        bb = xb_ref[:, b * 64:(b + 1) * 64] * scale_b + shift_b
        f = jnp.maximum(f, 0.0)
        bb = jnp.maximum(bb, 0.0)
        y = jnp.concatenate([f, bb], axis=-1)
        o_ref[b] = jnp.where(y > 0, y, 0.01 * y)


def _finish(outagg, gamma_f, beta_f, gamma_b, beta_b, b_f, b_b):
    sums, sqs = pl.pallas_call(
        _stats_kernel,
        grid=(2 * N // RD,),
        in_specs=[pl.BlockSpec((RD, 256), lambda i: (i, 0))],
        out_specs=[
            pl.BlockSpec((8, 256), lambda i: ((i * RD) // N, 0)),
            pl.BlockSpec((8, 256), lambda i: ((i * RD) // N, 0)),
        ],
        out_shape=[
            jax.ShapeDtypeStruct((16, 256), jnp.float32),
            jax.ShapeDtypeStruct((16, 256), jnp.float32),
        ],
    )(outagg)
    sums = sums.reshape(2, 8, 256)[:, 0, :]
    sqs = sqs.reshape(2, 8, 256)[:, 0, :]
    cnt = BATCH * N
    del b_f, b_b
    mean = sums.reshape(2, 4, 64).sum(axis=1) / cnt
    ex2 = sqs.reshape(2, 4, 64).sum(axis=1) / cnt
    var = ex2 - mean * mean
    rstd = lax.rsqrt(var + 1e-5)
    gam = jnp.stack([gamma_f, gamma_b], axis=0)
    bet = jnp.stack([beta_f, beta_b], axis=0)
    scale = rstd * gam
    shift2 = bet - mean * scale
    return pl.pallas_call(
        _finish_kernel,
        grid=(N // RN,),
        in_specs=[
            pl.BlockSpec((RN, 256), lambda i: (i, 0)),
            pl.BlockSpec((RN, 256), lambda i: (i + N // RN, 0)),
            pl.BlockSpec((2, 64), lambda i: (0, 0)),
            pl.BlockSpec((2, 64), lambda i: (0, 0)),
        ],
        out_specs=pl.BlockSpec((BATCH, RN, 128), lambda i: (0, i, 0)),
        out_shape=jax.ShapeDtypeStruct((BATCH, N, 128), jnp.float32),
    )(outagg, outagg, scale, shift2)


def kernel(data, org_edge_index, emb_f, emb_b, W_f, b_f, att_i_f, att_j_f,
           gamma_f, beta_f, W_b, b_b, att_i_b, att_j_b, gamma_b, beta_b):
    xtabf, xtabb, aif, aib = _build_tables(
        data, emb_f, emb_b, W_f, att_i_f, att_j_f, W_b, att_i_b, att_j_b)
    idx_f = _topk_idx(emb_f)
    of = _sc_aggregate_dir(xtabf, aif, idx_f)
    idx_b = _topk_idx(emb_b)
    ob = _sc_aggregate_dir(xtabb, aib, idx_b)
    outagg = jnp.concatenate(
        [of.reshape(NSC, 256)[:N], ob.reshape(NSC, 256)[:N]], axis=0)
    return _finish(outagg, gamma_f, beta_f, gamma_b, beta_b, b_f, b_b)


# speedup vs baseline: 1.8661x; 1.0009x over previous
import functools

import jax
import jax.numpy as jnp
from jax import lax
from jax.experimental import pallas as pl
from jax.experimental.pallas import tpu as pltpu
from jax.experimental.pallas import tpu_sc as plsc

N = 10000
DIM = 64
TOPK = 32
BATCH = 4
NPAD = 10240
NSLOT = 8
RA = 1000
RB = 1000
RD = 1000
RN = 400
NSC = 10240
NW = 32
PER_W = NSC // NW
G = 4
NGRP = PER_W // G


def _topk_kernel(embT_ref, er_ref, idx_ref):
    R = er_ref.shape[0]
    er = er_ref[...]
    vz = jnp.full((R, 128), -3.0, jnp.float32)
    iz = jnp.zeros((R, 128), jnp.int32)

    def superchunk(c, carry):
        vs = list(carry[:NSLOT])
        js = list(carry[NSLOT:])
        cs = c * 1024
        et = embT_ref[:, pl.ds(cs, 1024)]
        s = lax.dot_general(er, et, (((1,), (0,)), ((), ())),
                            preferred_element_type=jnp.float32)
        nsq = jnp.sum(et * et, axis=0, keepdims=True)
        rn = lax.rsqrt(nsq + 1e-30)
        col = cs + lax.broadcasted_iota(jnp.int32, (R, 1024), 1)
        sv = jnp.where(col < N, s * rn, -2.0)

        for k in range(8):
            cv = sv[:, k * 128:(k + 1) * 128]
            ci = cs + k * 128 + lax.broadcasted_iota(jnp.int32, (R, 128), 1)
            for t in range(NSLOT):
                cond = cv > vs[t]
                nv = jnp.where(cond, cv, vs[t])
                ni = jnp.where(cond, ci, js[t])
                cv = jnp.where(cond, vs[t], cv)
                ci = jnp.where(cond, js[t], ci)
                vs[t] = nv
                js[t] = ni
        return tuple(vs) + tuple(js)

    carry = lax.fori_loop(0, NPAD // 1024, superchunk,
                          (vz,) * NSLOT + (iz,) * NSLOT)
    v = jnp.concatenate(carry[:NSLOT], axis=1)
    ia = jnp.concatenate(carry[NSLOT:], axis=1)

    lk = lax.broadcasted_iota(jnp.int32, (R, TOPK), 1)

    def ext(k, carry2):
        v2, res = carry2
        m = jnp.max(v2, axis=1, keepdims=True)
        eq = v2 == m
        isel = jnp.where(eq, ia, jnp.int32(2**30))
        imin = jnp.min(isel, axis=1, keepdims=True)
        v2 = jnp.where(ia == imin, -3.0, v2)
        return (v2, jnp.where(lk == k, imin, res))

    _, res = lax.fori_loop(0, TOPK, ext,
                           (v, jnp.zeros((R, TOPK), jnp.int32)))
    idx_ref[...] = res


def _topk_idx(emb):
    embT = jnp.zeros((DIM, NPAD), jnp.float32).at[:, :N].set(emb.T)
    return pl.pallas_call(
        _topk_kernel,
        grid=(N // RA,),
        in_specs=[
            pl.BlockSpec((DIM, NPAD), lambda i: (0, 0)),
            pl.BlockSpec((RA, DIM), lambda i: (i, 0)),
        ],
        out_specs=pl.BlockSpec((RA, TOPK), lambda i: (i, 0)),
        out_shape=jax.ShapeDtypeStruct((N, TOPK), jnp.int32),
    )(embT, emb)


def _tables_kernel(data_ref, ef_ref, eb_ref, wu_ref, evf_ref, evb_ref,
                   xtf_ref, xtb_ref, aif_ref, aib_ref):
    R = ef_ref.shape[0]
    ef2 = jnp.dot(ef_ref[...], evf_ref[...],
                  preferred_element_type=jnp.float32)
    eb2 = jnp.dot(eb_ref[...], evb_ref[...],
                  preferred_element_type=jnp.float32)
    jf, if_, jb, ib = [], [], [], []
    for b in range(BATCH):
        t = jnp.dot(data_ref[b], wu_ref[...],
                    preferred_element_type=jnp.float32)
        xtf_ref[:, b * 64:(b + 1) * 64] = t[:, :64]
        xtb_ref[:, b * 64:(b + 1) * 64] = t[:, 64:128]
        jf.append(t[:, 128:129] + ef2[:, 0:1])
        if_.append(t[:, 129:130] + ef2[:, 1:2])
        jb.append(t[:, 130:131] + eb2[:, 0:1])
        ib.append(t[:, 131:132] + eb2[:, 1:2])
    z12 = jnp.zeros((R, 12), jnp.float32)
    xtf_ref[:, 256:272] = jnp.concatenate(jf + [z12], axis=1)
    xtb_ref[:, 256:272] = jnp.concatenate(jb + [z12], axis=1)
    aif_ref[...] = jnp.concatenate(if_ + [z12], axis=1)
    aib_ref[...] = jnp.concatenate(ib + [z12], axis=1)


def _build_tables(data, emb_f, emb_b, W_f, att_i_f, att_j_f,
                  W_b, att_i_b, att_j_b):
    Wbf = jnp.flip(W_b, axis=0)
    wu = jnp.zeros((10, 256), jnp.float32)
    wu = wu.at[:, :64].set(W_f).at[:, 64:128].set(Wbf)
    wu = wu.at[:, 128].set(W_f @ att_j_f[:DIM])
    wu = wu.at[:, 129].set(W_f @ att_i_f[:DIM])
    wu = wu.at[:, 130].set(Wbf @ att_j_b[:DIM])
    wu = wu.at[:, 131].set(Wbf @ att_i_b[:DIM])
    evf = jnp.stack([att_j_f[DIM:], att_i_f[DIM:]], axis=1)
    evb = jnp.stack([att_j_b[DIM:], att_i_b[DIM:]], axis=1)
    return pl.pallas_call(
        _tables_kernel,
        grid=(N // RB,),
        in_specs=[
            pl.BlockSpec((BATCH, RB, 10), lambda i: (0, i, 0)),
            pl.BlockSpec((RB, DIM), lambda i: (i, 0)),
            pl.BlockSpec((RB, DIM), lambda i: (i, 0)),
            pl.BlockSpec((10, 256), lambda i: (0, 0)),
            pl.BlockSpec((DIM, 2), lambda i: (0, 0)),
            pl.BlockSpec((DIM, 2), lambda i: (0, 0)),
        ],
        out_specs=[
            pl.BlockSpec((RB, 384), lambda i: (i, 0)),
            pl.BlockSpec((RB, 384), lambda i: (i, 0)),
            pl.BlockSpec((RB, 16), lambda i: (i, 0)),
            pl.BlockSpec((RB, 16), lambda i: (i, 0)),
        ],
        out_shape=[
            jax.ShapeDtypeStruct((NSC, 384), jnp.float32),
            jax.ShapeDtypeStruct((NSC, 384), jnp.float32),
            jax.ShapeDtypeStruct((NSC, 16), jnp.float32),
            jax.ShapeDtypeStruct((NSC, 16), jnp.float32),
        ],
    )(data, emb_f, emb_b, wu, evf, evb)


def _lane_bcast(v, lane):
    dn = lax.GatherDimensionNumbers(offset_dims=(), collapsed_slice_dims=(0,),
                                    start_index_map=(0,))
    idx = jnp.full((16, 1), lane, jnp.int32)
    return lax.gather(v, idx, dn, (1,),
                      mode=lax.GatherScatterMode.PROMISE_IN_BOUNDS)


def _sc_agg_body(xt, ai, idx, out_hbm,
                 idx_v, ai_v, rows_v, al_v, ostg_v, sem1):
    w = lax.axis_index("c") * 16 + lax.axis_index("s")
    pltpu.sync_copy(idx.at[pl.ds(w * PER_W * TOPK, PER_W * TOPK)], idx_v)
    pltpu.sync_copy(ai.at[pl.ds(w * PER_W * 16, PER_W * 16)], ai_v)

    def group(g, _):
        ixs = idx_v.at[pl.ds(g * (G * TOPK), G * TOPK)]
        pltpu.async_copy(xt.at[ixs], rows_v, sem1).wait()

        for ii in range(G):
            r0 = ii * TOPK
            nl = g * G + ii
            ai_vec = ai_v[pl.ds(nl * 16, 16)]

            def p1(k, m):
                alk = ai_vec + rows_v[r0 + k, pl.ds(256, 16)]
                alk = jnp.where(alk > 0, alk, 0.2 * alk)
                al_v[pl.ds(k * 16, 16)] = alk
                return jnp.maximum(m, alk)

            m = lax.fori_loop(0, TOPK, p1,
                              jnp.full((16,), -3.0e38, jnp.float32))

            def p2(k, den):
                e = jnp.exp(al_v[pl.ds(k * 16, 16)] - m)
                al_v[pl.ds(k * 16, 16)] = e
                return den + e

            den = lax.fori_loop(0, TOPK, p2, jnp.zeros((16,), jnp.float32))
            rden = 1.0 / (den + 1e-16)

            def p3(k, accs):
                wt = al_v[pl.ds(k * 16, 16)] * rden
                accs = list(accs)
                for b in range(BATCH):
                    wb = _lane_bcast(wt, b)
                    boff = b * 64
                    for cc in range(4):
                        accs[b * 4 + cc] = accs[b * 4 + cc] + wb * rows_v[
                            r0 + k, pl.ds(boff + cc * 16, 16)]
                return tuple(accs)

            z = jnp.zeros((16,), jnp.float32)
            accs = lax.fori_loop(0, TOPK, p3, (z,) * 16)
            for b in range(BATCH):
                for cc in range(4):
                    ostg_v[pl.ds(ii * 256 + b * 64 + cc * 16, 16)] = (
                        accs[b * 4 + cc])

        base = (w * PER_W + g * G) * 256
        pltpu.sync_copy(ostg_v, out_hbm.at[pl.ds(base, G * 256)])
        return 0
    lax.fori_loop(0, NGRP, group, 0)


def _sc_aggregate_dir(xtab, ai, idx):
    mesh = plsc.VectorSubcoreMesh(core_axis_name="c", subcore_axis_name="s")
    kern = functools.partial(
        pl.kernel,
        out_type=jax.ShapeDtypeStruct((NSC * 256,), jnp.float32),
        mesh=mesh,
        scratch_types=[
            pltpu.VMEM((PER_W * TOPK,), jnp.int32),
            pltpu.VMEM((PER_W * 16,), jnp.float32),
            pltpu.VMEM((G * TOPK, 384), jnp.float32),
            pltpu.VMEM((TOPK * 16,), jnp.float32),
            pltpu.VMEM((G * 256,), jnp.float32),
            pltpu.SemaphoreType.DMA,
        ],
    )(_sc_agg_body)
    idx_p = jnp.zeros((NSC, TOPK), jnp.int32).at[:N].set(idx)
    return kern(xtab, ai.reshape(NSC * 16), idx_p.reshape(NSC * TOPK))


def _stats_kernel(x_ref, sum_ref, sq_ref):
    i = pl.program_id(0)
    x = x_ref[...]
    pad = jnp.zeros((7, 256), jnp.float32)
    cs = jnp.concatenate([jnp.sum(x, axis=0, keepdims=True), pad], axis=0)
    cq = jnp.concatenate([jnp.sum(x * x, axis=0, keepdims=True), pad], axis=0)
    first = (i % (N // RD)) == 0

    @pl.when(first)
    def _():
        sum_ref[...] = cs
        sq_ref[...] = cq

    @pl.when(jnp.logical_not(first))
    def _():
        sum_ref[...] = sum_ref[...] + cs
        sq_ref[...] = sq_ref[...] + cq


def _finish_kernel(xf_ref, xb_ref, sc_ref, sh_ref, o_ref):
    scale_f = sc_ref[0:1, :]
    scale_b = sc_ref[1:2, :]
    shift_f = sh_ref[0:1, :]
    shift_b = sh_ref[1:2, :]
    for b in range(BATCH):
        f = xf_ref[:, b * 64:(b + 1) * 64] * scale_f + shift_f
        bb = xb_ref[:, b * 64:(b + 1) * 64] * scale_b + shift_b
        f = jnp.maximum(f, 0.0)
        bb = jnp.maximum(bb, 0.0)
        y = jnp.concatenate([f, bb], axis=-1)
        o_ref[b] = jnp.where(y > 0, y, 0.01 * y)


def _finish(outagg, gamma_f, beta_f, gamma_b, beta_b, b_f, b_b):
    sums, sqs = pl.pallas_call(
        _stats_kernel,
        grid=(2 * N // RD,),
        in_specs=[pl.BlockSpec((RD, 256), lambda i: (i, 0))],
        out_specs=[
            pl.BlockSpec((8, 256), lambda i: ((i * RD) // N, 0)),
            pl.BlockSpec((8, 256), lambda i: ((i * RD) // N, 0)),
        ],
        out_shape=[
            jax.ShapeDtypeStruct((16, 256), jnp.float32),
            jax.ShapeDtypeStruct((16, 256), jnp.float32),
        ],
    )(outagg)
    sums = sums.reshape(2, 8, 256)[:, 0, :]
    sqs = sqs.reshape(2, 8, 256)[:, 0, :]
    cnt = BATCH * N
    del b_f, b_b
    mean = sums.reshape(2, 4, 64).sum(axis=1) / cnt
    ex2 = sqs.reshape(2, 4, 64).sum(axis=1) / cnt
    var = ex2 - mean * mean
    rstd = lax.rsqrt(var + 1e-5)
    gam = jnp.stack([gamma_f, gamma_b], axis=0)
    bet = jnp.stack([beta_f, beta_b], axis=0)
    scale = rstd * gam
    shift2 = bet - mean * scale
    return pl.pallas_call(
        _finish_kernel,
        grid=(N // RN,),
        in_specs=[
            pl.BlockSpec((RN, 256), lambda i: (i, 0)),
            pl.BlockSpec((RN, 256), lambda i: (i + N // RN, 0)),
            pl.BlockSpec((2, 64), lambda i: (0, 0)),
            pl.BlockSpec((2, 64), lambda i: (0, 0)),
        ],
        out_specs=pl.BlockSpec((BATCH, RN, 128), lambda i: (0, i, 0)),
        out_shape=jax.ShapeDtypeStruct((BATCH, N, 128), jnp.float32),
    )(outagg, outagg, scale, shift2)


def kernel(data, org_edge_index, emb_f, emb_b, W_f, b_f, att_i_f, att_j_f,
           gamma_f, beta_f, W_b, b_b, att_i_b, att_j_b, gamma_b, beta_b):
    xtabf, xtabb, aif, aib = _build_tables(
        data, emb_f, emb_b, W_f, att_i_f, att_j_f, W_b, att_i_b, att_j_b)
    idx_f = _topk_idx(emb_f)
    of = _sc_aggregate_dir(xtabf, aif, idx_f)
    idx_b = _topk_idx(emb_b)
    ob = _sc_aggregate_dir(xtabb, aib, idx_b)
    outagg = jnp.concatenate(
        [of.reshape(NSC, 256)[:N], ob.reshape(NSC, 256)[:N]], axis=0)
    return _finish(outagg, gamma_f, beta_f, gamma_b, beta_b, b_f, b_b)
